# Initial kernel scaffold; baseline (speedup 1.0000x reference)
import jax
import jax.numpy as jnp
from jax.experimental import pallas as pl


def kernel(token_ids, embedding):
    raise NotImplementedError("write your pallas kernel here")


# speedup vs baseline: 3.3318x; 3.3318x over previous
import functools

import jax
import jax.numpy as jnp
from jax import lax
from jax.experimental import pallas as pl
from jax.experimental.pallas import tpu as pltpu
from jax.experimental.pallas import tpu_sc as plsc

D = 128
B = 4096 * 50
NC, NS = 2, 16
NW = NC * NS
B_PER_W = B // NW
C = 128
NCHUNK = B_PER_W // C
NBUF = 2
NGROUP = NCHUNK // NBUF

_mesh = plsc.VectorSubcoreMesh(core_axis_name="c", subcore_axis_name="s")


@functools.partial(
    pl.kernel,
    mesh=_mesh,
    out_type=jax.ShapeDtypeStruct((B, D), jnp.float32),
    scratch_types=[
        pltpu.VMEM((B_PER_W,), jnp.int32),
        pltpu.VMEM((NBUF, C, D), jnp.float32),
        pltpu.SemaphoreType.DMA,
        pltpu.SemaphoreType.DMA,
    ],
)
def _emb_lookup(idx_hbm, table_hbm, out_hbm, idx_v, rows_v, sem0, sem1):
    sems = [sem0, sem1]
    wid = lax.axis_index("s") * NC + lax.axis_index("c")
    base = wid * B_PER_W

    pltpu.sync_copy(idx_hbm.at[pl.ds(base, B_PER_W)], idx_v)

    for b in range(NBUF):
        pltpu.async_copy(
            table_hbm.at[idx_v.at[pl.ds(b * C, C)]], rows_v.at[b], sems[b]
        )

    def group(g, carry):
        for b in range(NBUF):
            i = g * NBUF + b
            pltpu.make_async_copy(
                table_hbm.at[idx_v.at[pl.ds(i * C, C)]], rows_v.at[b], sems[b]
            ).wait()
            pltpu.sync_copy(rows_v.at[b], out_hbm.at[pl.ds(base + i * C, C)])
            nxt = i + NBUF

            @pl.when(nxt < NCHUNK)
            def _():
                pltpu.async_copy(
                    table_hbm.at[idx_v.at[pl.ds(nxt * C, C)]],
                    rows_v.at[b],
                    sems[b],
                )

        return carry

    lax.fori_loop(0, NGROUP, group, 0)


def kernel(token_ids, embedding):
    flat = token_ids.reshape(-1).astype(jnp.int32)
    out = _emb_lookup(flat, embedding)
    return out.reshape(token_ids.shape + (embedding.shape[1],))
